# baseline (speedup 1.0000x reference)
import functools

import jax
import jax.numpy as jnp
from jax import lax
from jax.experimental import pallas as pl
from jax.experimental.pallas import tpu as pltpu
from jax.experimental.pallas import tpu_sc as plsc

H = 64
W = 64
HW = H * W
ITERATIONS = 3
RADIUS = 4
NDRAW = ITERATIONS * RADIUS
NW = 32
RPW = (2 * H) // NW
LANES = 16
MAGIC = 8388608.0


def _dfield_body(s_ref, t_ref, o_ref):
    c = pl.program_id(1)
    nc = pl.num_programs(1)
    diff = s_ref[0] - t_ref[0]
    part = jnp.sum(diff * diff, axis=0)

    @pl.when(c == 0)
    def _():
        o_ref[0] = part

    @pl.when(c != 0)
    def _():
        o_ref[0] = o_ref[0] + part

    @pl.when(c == nc - 1)
    def _():
        e2 = o_ref[0]
        rowi = lax.broadcasted_iota(jnp.int32, (H, W), 0)
        coli = lax.broadcasted_iota(jnp.int32, (H, W), 1)
        left = jnp.where(coli >= 1, jnp.roll(e2, 1, axis=1), 0.0)
        right = jnp.where(coli <= W - 2, jnp.roll(e2, -1, axis=1), 0.0)
        sx = e2 + left + right
        up = jnp.where(rowi >= 1, jnp.roll(sx, 1, axis=0), 0.0)
        down = jnp.where(rowi <= H - 2, jnp.roll(sx, -1, axis=0), 0.0)
        o_ref[0] = sx + up + down


def _dfield(source, target):
    B, C, _, _ = source.shape
    CB = 32
    return pl.pallas_call(
        _dfield_body,
        grid=(B, C // CB),
        in_specs=[
            pl.BlockSpec((1, CB, H, W), lambda b, c: (b, c, 0, 0)),
            pl.BlockSpec((1, CB, H, W), lambda b, c: (b, c, 0, 0)),
        ],
        out_specs=pl.BlockSpec((1, H, W), lambda b, c: (b, 0, 0)),
        out_shape=jax.ShapeDtypeStruct((B, H, W), jnp.float32),
    )(source, target)


def _sc_loop_body(d_hbm, u_hbm, o_hbm, out_hbm, uv, ov, dv, yb, xb, db, sem):
    wid = lax.axis_index("s") * 2 + lax.axis_index("c")
    b = wid // (NW // 2)
    h0 = (wid - b * (NW // 2)) * RPW

    copies = []
    for dr in range(NDRAW):
        for comp in range(2):
            src = o_hbm.at[dr, pl.ds((b * 2 + comp) * HW + h0 * W, RPW * W)]
            dst = ov.at[pl.ds((dr * 2 + comp) * RPW * W, RPW * W)]
            copies.append(pltpu.async_copy(src, dst, sem))
    for comp in range(2):
        src = u_hbm.at[pl.ds((b * 2 + comp) * HW + h0 * W, RPW * W)]
        copies.append(pltpu.async_copy(src, uv.at[pl.ds(comp * RPW * W, RPW * W)], sem))
    copies.append(pltpu.async_copy(d_hbm.at[pl.ds(b * HW, HW)], dv, sem))
    for cp in copies:
        cp.wait()

    iota = lax.iota(jnp.int32, LANES)
    nchunk = W // LANES

    def lg(ref, idx):
        return plsc.load_gather(ref, [idx])

    def rnd_int(v):
        return ((v + MAGIC) - MAGIC).astype(jnp.int32)

    def row_body(r, carry):
        base = r * W

        for i in range(nchunk):
            ci = iota + (base + i * LANES)
            uy = lg(uv, iota + (r * W + i * LANES))
            ux = lg(uv, iota + (RPW * W + r * W + i * LANES))
            y = uy * float(H - 1)
            x = ux * float(W - 1)
            d0 = lg(dv, rnd_int(y) * W + rnd_int(x))
            plsc.store_scatter(yb, [ci], y)
            plsc.store_scatter(xb, [ci], x)
            plsc.store_scatter(db, [ci], d0)

        def t_body(t, tc):
            for dirn in (1, -1):
                res = []
                for i in range(nchunk):
                    ci = iota + (base + i * LANES)
                    sj = ((iota + (i * LANES - dirn)) & (W - 1)) + base
                    cy, cx, cd = lg(yb, ci), lg(xb, ci), lg(db, ci)
                    sy, sx, sd = lg(yb, sj), lg(xb, sj), lg(db, sj)
                    m = sd < cd
                    res.append((ci,
                                jnp.where(m, sy, cy),
                                jnp.where(m, sx, cx),
                                jnp.where(m, sd, cd)))
                for ci, ny, nx, nd in res:
                    plsc.store_scatter(yb, [ci], ny)
                    plsc.store_scatter(xb, [ci], nx)
                    plsc.store_scatter(db, [ci], nd)

            for s in range(RADIUS):
                dr = t * RADIUS + s
                sc = 2.0 ** (-s)
                for i in range(nchunk):
                    ci = iota + (base + i * LANES)
                    ob = dr * 2 * RPW * W + r * W + i * LANES
                    offy = lg(ov, iota + ob) * sc
                    offx = lg(ov, iota + (ob + RPW * W)) * sc
                    y, x, d0 = lg(yb, ci), lg(xb, ci), lg(db, ci)
                    ry = jnp.minimum(jnp.maximum(y + offy, 0.0), float(H - 1))
                    rx = jnp.minimum(jnp.maximum(x + offx, 0.0), float(W - 1))
                    rd = lg(dv, rnd_int(ry) * W + rnd_int(rx))
                    m = rd < d0
                    plsc.store_scatter(yb, [ci], jnp.where(m, ry, y))
                    plsc.store_scatter(xb, [ci], jnp.where(m, rx, x))
                    plsc.store_scatter(db, [ci], jnp.where(m, rd, d0))
            return tc

        lax.fori_loop(0, ITERATIONS, t_body, 0)

        for i in range(nchunk):
            ci = iota + (base + i * LANES)
            plsc.store_scatter(uv, [iota + (r * W + i * LANES)], lg(yb, ci))
            plsc.store_scatter(uv, [iota + (RPW * W + r * W + i * LANES)], lg(xb, ci))
        return carry

    lax.fori_loop(0, RPW, row_body, 0)

    for comp in range(2):
        pltpu.sync_copy(uv.at[pl.ds(comp * RPW * W, RPW * W)],
                        out_hbm.at[pl.ds((b * 2 + comp) * HW + h0 * W, RPW * W)])


def _sc_loop(d_flat, u_flat, offs2):
    B = 2
    mesh = plsc.VectorSubcoreMesh(core_axis_name="c", subcore_axis_name="s")
    fn = functools.partial(
        pl.kernel,
        mesh=mesh,
        out_type=jax.ShapeDtypeStruct((B * 2 * HW,), jnp.float32),
        scratch_types=[
            pltpu.VMEM((2 * RPW * W,), jnp.float32),
            pltpu.VMEM((NDRAW * 2 * RPW * W,), jnp.float32),
            pltpu.VMEM((HW,), jnp.float32),
            pltpu.VMEM((RPW * W,), jnp.float32),
            pltpu.VMEM((RPW * W,), jnp.float32),
            pltpu.VMEM((RPW * W,), jnp.float32),
            pltpu.SemaphoreType.DMA,
        ],
        compiler_params=pltpu.CompilerParams(needs_layout_passes=False),
    )(_sc_loop_body)
    return fn(d_flat, u_flat, offs2)


_K0 = (64467757, 2916123636)
_KS = [[2451885785, 2215112154], [2477523575, 3040475525],
       [3288317168, 3869482587], [3554626980, 3142212981],
       [1115580475, 397968394], [3965541470, 1466314410],
       [1329917820, 631477198], [3389937870, 4222981018],
       [845657194, 2085162261], [2019228077, 1846897043],
       [1878397639, 3912187480], [3118403341, 2122305751]]


def kernel(source, target):
    B, C, _, _ = source.shape

    dfield = _dfield(source, target)

    k0 = jax.random.wrap_key_data(jnp.array(_K0, dtype=jnp.uint32))
    u = jax.random.uniform(k0, (B * 2 * HW,), dtype=jnp.float32)
    ks = jnp.array(_KS, dtype=jnp.uint32)
    offs = jax.vmap(
        lambda kk: jax.random.normal(jax.random.wrap_key_data(kk),
                                     (B * 2 * HW,), dtype=jnp.float32))(ks)

    out_flat = _sc_loop(dfield.reshape(-1), u, offs)
    return out_flat.reshape(B, 2, H, W)


# speedup vs baseline: 1.9130x; 1.9130x over previous
import functools

import jax
import jax.numpy as jnp
from jax import lax
from jax.experimental import pallas as pl
from jax.experimental.pallas import tpu as pltpu
from jax.experimental.pallas import tpu_sc as plsc

H = 64
W = 64
HW = H * W
ITERATIONS = 3
RADIUS = 4
NDRAW = ITERATIONS * RADIUS
NW = 32
RPW = (2 * H) // NW
LANES = 16
MAGIC = 8388608.0


_HB = 16


def _dfield_body(s_ref, t_ref, o_ref, e2_ref):
    s = pl.program_id(1)
    ns = pl.num_programs(1)
    diff = s_ref[0] - t_ref[0]
    e2_ref[pl.ds(s * _HB, _HB), :] = jnp.sum(diff * diff, axis=-1)

    @pl.when(s == ns - 1)
    def _():
        e2 = e2_ref[...]
        rowi = lax.broadcasted_iota(jnp.int32, (H, W), 0)
        coli = lax.broadcasted_iota(jnp.int32, (H, W), 1)
        left = jnp.where(coli >= 1, jnp.roll(e2, 1, axis=1), 0.0)
        right = jnp.where(coli <= W - 2, jnp.roll(e2, -1, axis=1), 0.0)
        sx = e2 + left + right
        up = jnp.where(rowi >= 1, jnp.roll(sx, 1, axis=0), 0.0)
        down = jnp.where(rowi <= H - 2, jnp.roll(sx, -1, axis=0), 0.0)
        o_ref[0] = sx + up + down


def _dfield(source, target):
    B, C, _, _ = source.shape
    st = jnp.transpose(source, (0, 2, 3, 1))
    tt = jnp.transpose(target, (0, 2, 3, 1))
    return pl.pallas_call(
        _dfield_body,
        grid=(B, H // _HB),
        in_specs=[
            pl.BlockSpec((1, _HB, W, C), lambda b, s: (b, s, 0, 0)),
            pl.BlockSpec((1, _HB, W, C), lambda b, s: (b, s, 0, 0)),
        ],
        out_specs=pl.BlockSpec((1, H, W), lambda b, s: (b, 0, 0)),
        out_shape=jax.ShapeDtypeStruct((B, H, W), jnp.float32),
        scratch_shapes=[pltpu.VMEM((H, W), jnp.float32)],
    )(st, tt)


def _sc_loop_body(d_hbm, u_hbm, o_hbm, out_hbm, uv, ov, dv, yb, xb, db, sem):
    wid = lax.axis_index("s") * 2 + lax.axis_index("c")
    b = wid // (NW // 2)
    h0 = (wid - b * (NW // 2)) * RPW

    copies = []
    for dr in range(NDRAW):
        for comp in range(2):
            src = o_hbm.at[dr, pl.ds((b * 2 + comp) * HW + h0 * W, RPW * W)]
            dst = ov.at[pl.ds((dr * 2 + comp) * RPW * W, RPW * W)]
            copies.append(pltpu.async_copy(src, dst, sem))
    for comp in range(2):
        src = u_hbm.at[pl.ds((b * 2 + comp) * HW + h0 * W, RPW * W)]
        copies.append(pltpu.async_copy(src, uv.at[pl.ds(comp * RPW * W, RPW * W)], sem))
    copies.append(pltpu.async_copy(d_hbm.at[pl.ds(b * HW, HW)], dv, sem))
    for cp in copies:
        cp.wait()

    iota = lax.iota(jnp.int32, LANES)
    nchunk = W // LANES

    def lg(ref, idx):
        return plsc.load_gather(ref, [idx])

    def rnd_int(v):
        return ((v + MAGIC) - MAGIC).astype(jnp.int32)

    def row_body(r, carry):
        base = r * W

        for i in range(nchunk):
            ci = iota + (base + i * LANES)
            uy = lg(uv, iota + (r * W + i * LANES))
            ux = lg(uv, iota + (RPW * W + r * W + i * LANES))
            y = uy * float(H - 1)
            x = ux * float(W - 1)
            d0 = lg(dv, rnd_int(y) * W + rnd_int(x))
            plsc.store_scatter(yb, [ci], y)
            plsc.store_scatter(xb, [ci], x)
            plsc.store_scatter(db, [ci], d0)

        def t_body(t, tc):
            for dirn in (1, -1):
                res = []
                for i in range(nchunk):
                    ci = iota + (base + i * LANES)
                    sj = ((iota + (i * LANES - dirn)) & (W - 1)) + base
                    cy, cx, cd = lg(yb, ci), lg(xb, ci), lg(db, ci)
                    sy, sx, sd = lg(yb, sj), lg(xb, sj), lg(db, sj)
                    m = sd < cd
                    res.append((ci,
                                jnp.where(m, sy, cy),
                                jnp.where(m, sx, cx),
                                jnp.where(m, sd, cd)))
                for ci, ny, nx, nd in res:
                    plsc.store_scatter(yb, [ci], ny)
                    plsc.store_scatter(xb, [ci], nx)
                    plsc.store_scatter(db, [ci], nd)

            for s in range(RADIUS):
                dr = t * RADIUS + s
                sc = 2.0 ** (-s)
                for i in range(nchunk):
                    ci = iota + (base + i * LANES)
                    ob = dr * 2 * RPW * W + r * W + i * LANES
                    offy = lg(ov, iota + ob) * sc
                    offx = lg(ov, iota + (ob + RPW * W)) * sc
                    y, x, d0 = lg(yb, ci), lg(xb, ci), lg(db, ci)
                    ry = jnp.minimum(jnp.maximum(y + offy, 0.0), float(H - 1))
                    rx = jnp.minimum(jnp.maximum(x + offx, 0.0), float(W - 1))
                    rd = lg(dv, rnd_int(ry) * W + rnd_int(rx))
                    m = rd < d0
                    plsc.store_scatter(yb, [ci], jnp.where(m, ry, y))
                    plsc.store_scatter(xb, [ci], jnp.where(m, rx, x))
                    plsc.store_scatter(db, [ci], jnp.where(m, rd, d0))
            return tc

        lax.fori_loop(0, ITERATIONS, t_body, 0)

        for i in range(nchunk):
            ci = iota + (base + i * LANES)
            plsc.store_scatter(uv, [iota + (r * W + i * LANES)], lg(yb, ci))
            plsc.store_scatter(uv, [iota + (RPW * W + r * W + i * LANES)], lg(xb, ci))
        return carry

    lax.fori_loop(0, RPW, row_body, 0)

    for comp in range(2):
        pltpu.sync_copy(uv.at[pl.ds(comp * RPW * W, RPW * W)],
                        out_hbm.at[pl.ds((b * 2 + comp) * HW + h0 * W, RPW * W)])


def _sc_loop(d_flat, u_flat, offs2):
    B = 2
    mesh = plsc.VectorSubcoreMesh(core_axis_name="c", subcore_axis_name="s")
    fn = functools.partial(
        pl.kernel,
        mesh=mesh,
        out_type=jax.ShapeDtypeStruct((B * 2 * HW,), jnp.float32),
        scratch_types=[
            pltpu.VMEM((2 * RPW * W,), jnp.float32),
            pltpu.VMEM((NDRAW * 2 * RPW * W,), jnp.float32),
            pltpu.VMEM((HW,), jnp.float32),
            pltpu.VMEM((RPW * W,), jnp.float32),
            pltpu.VMEM((RPW * W,), jnp.float32),
            pltpu.VMEM((RPW * W,), jnp.float32),
            pltpu.SemaphoreType.DMA,
        ],
        compiler_params=pltpu.CompilerParams(needs_layout_passes=False),
    )(_sc_loop_body)
    return fn(d_flat, u_flat, offs2)


_K0 = (64467757, 2916123636)
_KS = [[2451885785, 2215112154], [2477523575, 3040475525],
       [3288317168, 3869482587], [3554626980, 3142212981],
       [1115580475, 397968394], [3965541470, 1466314410],
       [1329917820, 631477198], [3389937870, 4222981018],
       [845657194, 2085162261], [2019228077, 1846897043],
       [1878397639, 3912187480], [3118403341, 2122305751]]


def kernel(source, target):
    B, C, _, _ = source.shape

    dfield = _dfield(source, target)

    k0 = jax.random.wrap_key_data(jnp.array(_K0, dtype=jnp.uint32))
    u = jax.random.uniform(k0, (B * 2 * HW,), dtype=jnp.float32)
    ks = jnp.array(_KS, dtype=jnp.uint32)
    offs = jax.vmap(
        lambda kk: jax.random.normal(jax.random.wrap_key_data(kk),
                                     (B * 2 * HW,), dtype=jnp.float32))(ks)

    out_flat = _sc_loop(dfield.reshape(-1), u, offs)
    return out_flat.reshape(B, 2, H, W)


# speedup vs baseline: 2.0044x; 1.0478x over previous
import functools

import jax
import jax.numpy as jnp
from jax import lax
from jax.experimental import pallas as pl
from jax.experimental.pallas import tpu as pltpu
from jax.experimental.pallas import tpu_sc as plsc

H = 64
W = 64
HW = H * W
ITERATIONS = 3
RADIUS = 4
NDRAW = ITERATIONS * RADIUS
NW = 32
RPW = (2 * H) // NW
LANES = 16
MAGIC = 8388608.0


_HB = 16


def _dfield_body(s_ref, t_ref, o_ref, e2_ref):
    s = pl.program_id(1)
    ns = pl.num_programs(1)
    diff = s_ref[0] - t_ref[0]
    e2_ref[pl.ds(s * _HB, _HB), :] = jnp.sum(diff * diff, axis=-1)

    @pl.when(s == ns - 1)
    def _():
        e2 = e2_ref[...]
        rowi = lax.broadcasted_iota(jnp.int32, (H, W), 0)
        coli = lax.broadcasted_iota(jnp.int32, (H, W), 1)
        left = jnp.where(coli >= 1, jnp.roll(e2, 1, axis=1), 0.0)
        right = jnp.where(coli <= W - 2, jnp.roll(e2, -1, axis=1), 0.0)
        sx = e2 + left + right
        up = jnp.where(rowi >= 1, jnp.roll(sx, 1, axis=0), 0.0)
        down = jnp.where(rowi <= H - 2, jnp.roll(sx, -1, axis=0), 0.0)
        o_ref[0] = sx + up + down


def _dfield(source, target):
    B, C, _, _ = source.shape
    st = jnp.transpose(source, (0, 2, 3, 1))
    tt = jnp.transpose(target, (0, 2, 3, 1))
    return pl.pallas_call(
        _dfield_body,
        grid=(B, H // _HB),
        in_specs=[
            pl.BlockSpec((1, _HB, W, C), lambda b, s: (b, s, 0, 0)),
            pl.BlockSpec((1, _HB, W, C), lambda b, s: (b, s, 0, 0)),
        ],
        out_specs=pl.BlockSpec((1, H, W), lambda b, s: (b, 0, 0)),
        out_shape=jax.ShapeDtypeStruct((B, H, W), jnp.float32),
        scratch_shapes=[pltpu.VMEM((H, W), jnp.float32)],
    )(st, tt)


_ROTP = tuple([15] + list(range(15)))
_ROTM = tuple(list(range(1, 16)) + [0])


def _sc_loop_body(d_hbm, u_hbm, o_hbm, out_hbm, uv, ov, dv, sem_ud, sem_off):
    wid = lax.axis_index("s") * 2 + lax.axis_index("c")
    b = wid // (NW // 2)
    h0 = (wid - b * (NW // 2)) * RPW

    copies = []
    for dr in range(NDRAW):
        for comp in range(2):
            src = o_hbm.at[dr, pl.ds((b * 2 + comp) * HW + h0 * W, RPW * W)]
            dst = ov.at[pl.ds((dr * 2 + comp) * RPW * W, RPW * W)]
            copies.append(pltpu.async_copy(src, dst, sem_off))
    ud = []
    for comp in range(2):
        src = u_hbm.at[pl.ds((b * 2 + comp) * HW + h0 * W, RPW * W)]
        ud.append(pltpu.async_copy(src, uv.at[pl.ds(comp * RPW * W, RPW * W)], sem_ud))
    ud.append(pltpu.async_copy(d_hbm.at[pl.ds(b * HW, HW)], dv, sem_ud))
    for cp in ud:
        cp.wait()

    iota = lax.iota(jnp.int32, LANES)
    nchunk = W // LANES
    rotp_idx = (iota + (LANES - 1)) & (LANES - 1)
    rotm_idx = (iota + 1) & (LANES - 1)
    is0 = iota == 0
    is15 = iota == LANES - 1

    def lg(idx):
        return plsc.load_gather(dv, [idx])

    def rnd_int(v):
        return ((v + MAGIC) - MAGIC).astype(jnp.int32)

    def rot(v, idx):
        return v.at[idx].get(mode="promise_in_bounds")

    waited_offs = False
    for r in range(RPW):
        ys, xs, ds = [], [], []
        for i in range(nchunk):
            y = uv[pl.ds(r * W + i * LANES, LANES)] * float(H - 1)
            x = uv[pl.ds(RPW * W + r * W + i * LANES, LANES)] * float(W - 1)
            ys.append(y)
            xs.append(x)
            ds.append(lg(rnd_int(y) * W + rnd_int(x)))

        for t in range(ITERATIONS):
            for dirn in (1, -1):
                if dirn == 1:
                    ry = [rot(v, rotp_idx) for v in ys]
                    rx = [rot(v, rotp_idx) for v in xs]
                    rd = [rot(v, rotp_idx) for v in ds]
                    sy = [jnp.where(is0, ry[i - 1], ry[i]) for i in range(nchunk)]
                    sx = [jnp.where(is0, rx[i - 1], rx[i]) for i in range(nchunk)]
                    sd = [jnp.where(is0, rd[i - 1], rd[i]) for i in range(nchunk)]
                else:
                    ry = [rot(v, rotm_idx) for v in ys]
                    rx = [rot(v, rotm_idx) for v in xs]
                    rd = [rot(v, rotm_idx) for v in ds]
                    sy = [jnp.where(is15, ry[(i + 1) % nchunk], ry[i])
                          for i in range(nchunk)]
                    sx = [jnp.where(is15, rx[(i + 1) % nchunk], rx[i])
                          for i in range(nchunk)]
                    sd = [jnp.where(is15, rd[(i + 1) % nchunk], rd[i])
                          for i in range(nchunk)]
                for i in range(nchunk):
                    m = sd[i] < ds[i]
                    ys[i] = jnp.where(m, sy[i], ys[i])
                    xs[i] = jnp.where(m, sx[i], xs[i])
                    ds[i] = jnp.where(m, sd[i], ds[i])

            if not waited_offs:
                for cp in copies:
                    cp.wait()
                waited_offs = True
            for s in range(RADIUS):
                dr = t * RADIUS + s
                sc = 2.0 ** (-s)
                for i in range(nchunk):
                    ob = dr * 2 * RPW * W + r * W + i * LANES
                    offy = ov[pl.ds(ob, LANES)] * sc
                    offx = ov[pl.ds(ob + RPW * W, LANES)] * sc
                    cy = jnp.minimum(jnp.maximum(ys[i] + offy, 0.0), float(H - 1))
                    cx = jnp.minimum(jnp.maximum(xs[i] + offx, 0.0), float(W - 1))
                    cd = lg(rnd_int(cy) * W + rnd_int(cx))
                    m = cd < ds[i]
                    ys[i] = jnp.where(m, cy, ys[i])
                    xs[i] = jnp.where(m, cx, xs[i])
                    ds[i] = jnp.where(m, cd, ds[i])

        for i in range(nchunk):
            uv[pl.ds(r * W + i * LANES, LANES)] = ys[i]
            uv[pl.ds(RPW * W + r * W + i * LANES, LANES)] = xs[i]

    for comp in range(2):
        pltpu.sync_copy(uv.at[pl.ds(comp * RPW * W, RPW * W)],
                        out_hbm.at[pl.ds((b * 2 + comp) * HW + h0 * W, RPW * W)])


def _sc_loop(d_flat, u_flat, offs2):
    B = 2
    mesh = plsc.VectorSubcoreMesh(core_axis_name="c", subcore_axis_name="s")
    fn = functools.partial(
        pl.kernel,
        mesh=mesh,
        out_type=jax.ShapeDtypeStruct((B * 2 * HW,), jnp.float32),
        scratch_types=[
            pltpu.VMEM((2 * RPW * W,), jnp.float32),
            pltpu.VMEM((NDRAW * 2 * RPW * W,), jnp.float32),
            pltpu.VMEM((HW,), jnp.float32),
            pltpu.SemaphoreType.DMA,
            pltpu.SemaphoreType.DMA,
        ],
        compiler_params=pltpu.CompilerParams(needs_layout_passes=False),
    )(_sc_loop_body)
    return fn(d_flat, u_flat, offs2)


_K0 = (64467757, 2916123636)
_KS = [[2451885785, 2215112154], [2477523575, 3040475525],
       [3288317168, 3869482587], [3554626980, 3142212981],
       [1115580475, 397968394], [3965541470, 1466314410],
       [1329917820, 631477198], [3389937870, 4222981018],
       [845657194, 2085162261], [2019228077, 1846897043],
       [1878397639, 3912187480], [3118403341, 2122305751]]


def kernel(source, target):
    B, C, _, _ = source.shape

    dfield = _dfield(source, target)

    k0 = jax.random.wrap_key_data(jnp.array(_K0, dtype=jnp.uint32))
    u = jax.random.uniform(k0, (B * 2 * HW,), dtype=jnp.float32)
    ks = jnp.array(_KS, dtype=jnp.uint32)
    offs = jax.vmap(
        lambda kk: jax.random.normal(jax.random.wrap_key_data(kk),
                                     (B * 2 * HW,), dtype=jnp.float32))(ks)

    out_flat = _sc_loop(dfield.reshape(-1), u, offs)
    return out_flat.reshape(B, 2, H, W)


# speedup vs baseline: 2.3624x; 1.1786x over previous
import functools

import jax
import jax.numpy as jnp
import numpy as np
from jax import lax
from jax.experimental import pallas as pl
from jax.experimental.pallas import tpu as pltpu
from jax.experimental.pallas import tpu_sc as plsc

H = 64
W = 64
HW = H * W
ITERATIONS = 3
RADIUS = 4
NDRAW = ITERATIONS * RADIUS
NW = 32
RPW = (2 * H) // NW
LANES = 16
MAGIC = 8388608.0


_HB = 16

_ROT_A = (13, 15, 26, 6)
_ROT_B = (17, 29, 16, 24)
_LO = np.float32(np.nextafter(np.float32(-1.0), np.float32(0.0)))
_SPAN = np.float32(np.float32(1.0) - _LO)
_SQRT2 = np.float32(np.sqrt(2.0))


def _rotl(x, r):
    return (x << np.uint32(r)) | (x >> np.uint32(32 - r))


def _threefry_bits(k0, k1, x1):
    ks0 = np.uint32(k0)
    ks1 = np.uint32(k1)
    ks2 = np.uint32(np.uint32(0x1BD11BDA) ^ ks0 ^ ks1)
    ks = (ks0, ks1, ks2)
    x0 = jnp.full_like(x1, ks0)
    x1 = x1 + ks1
    for i in range(5):
        for r in (_ROT_A if i % 2 == 0 else _ROT_B):
            x0 = x0 + x1
            x1 = _rotl(x1, r)
            x1 = x1 ^ x0
        x0 = x0 + ks[(i + 1) % 3]
        x1 = x1 + np.uint32(ks[(i + 2) % 3] + np.uint32(i + 1))
    return x0 ^ x1


def _bits_to_unit(bits):
    f = lax.bitcast_convert_type((bits >> np.uint32(9)) | np.uint32(0x3F800000),
                                 jnp.float32)
    return f - 1.0


def _erfinv(x):
    w = -jnp.log((1.0 - x) * (1.0 + x))
    w1 = w - 2.5
    p1 = jnp.full_like(x, np.float32(2.81022636e-08))
    for c in (3.43273939e-07, -3.5233877e-06, -4.39150654e-06, 0.00021858087,
              -0.00125372503, -0.00417768164, 0.246640727, 1.50140941):
        p1 = np.float32(c) + p1 * w1
    w2 = jnp.sqrt(w) - 3.0
    p2 = jnp.full_like(x, np.float32(-0.000200214257))
    for c in (0.000100950558, 0.00134934322, -0.00367342844, 0.00573950773,
              -0.0076224613, 0.00943887047, 1.00167406, 2.83297682):
        p2 = np.float32(c) + p2 * w2
    return jnp.where(w < 5.0, p1, p2) * x


def _dfield_body(s_ref, t_ref, o_ref, u_ref, *offs_and_scratch):
    off_refs = offs_and_scratch[:NDRAW]
    e2_ref = offs_and_scratch[NDRAW]
    b = pl.program_id(0)
    s = pl.program_id(1)
    ns = pl.num_programs(1)

    k = b * ns + s
    iv = (lax.broadcasted_iota(jnp.uint32, (16, 128), 0) * np.uint32(128)
          + lax.broadcasted_iota(jnp.uint32, (16, 128), 1))
    iv = iv + (k * 2048).astype(jnp.uint32)
    u_ref[...] = jnp.maximum(0.0, _bits_to_unit(_threefry_bits(_K0[0], _K0[1], iv)))
    for dr in range(NDRAW):
        un = _bits_to_unit(_threefry_bits(_KS[dr][0], _KS[dr][1], iv))
        v = jnp.maximum(_LO, un * _SPAN + _LO)
        off_refs[dr][...] = _SQRT2 * _erfinv(v)

    hb = 32 // ns
    diff = s_ref[0] - t_ref[0]
    e2_ref[pl.ds(s * hb, hb), :] = jnp.sum(diff * diff, axis=-1)

    @pl.when(s == ns - 1)
    def _():
        e2 = e2_ref[...]
        rowi = lax.broadcasted_iota(jnp.int32, (32, 128), 0)
        coli = lax.broadcasted_iota(jnp.int32, (32, 128), 1)
        xi = coli & (W - 1)
        hi = coli >= W
        left = jnp.where(xi >= 1, jnp.roll(e2, 1, axis=1), 0.0)
        right = jnp.where(xi <= W - 2, jnp.roll(e2, -1, axis=1), 0.0)
        sx = e2 + left + right
        upA = jnp.roll(sx, W, axis=1)
        upC = jnp.roll(jnp.roll(sx, 1, axis=0), -W, axis=1)
        up = jnp.where(hi, upA, jnp.where(rowi >= 1, upC, 0.0))
        dnA = jnp.roll(sx, -W, axis=1)
        dnC = jnp.roll(jnp.roll(sx, -1, axis=0), W, axis=1)
        dn = jnp.where(hi, jnp.where(rowi <= 30, dnC, 0.0), dnA)
        o_ref[0] = sx + up + dn


def _dfield(source, target):
    B, C, _, _ = source.shape
    ns = 4
    st = jnp.transpose(source, (0, 2, 3, 1)).reshape(B, 32, 2 * W, C)
    tt = jnp.transpose(target, (0, 2, 3, 1)).reshape(B, 32, 2 * W, C)
    rng_spec = pl.BlockSpec((16, 128), lambda b, s: (b * 4 + s, 0))
    rng_shape = jax.ShapeDtypeStruct((128, 128), jnp.float32)
    return pl.pallas_call(
        _dfield_body,
        grid=(B, ns),
        in_specs=[
            pl.BlockSpec((1, 32 // ns, 2 * W, C), lambda b, s: (b, s, 0, 0)),
            pl.BlockSpec((1, 32 // ns, 2 * W, C), lambda b, s: (b, s, 0, 0)),
        ],
        out_specs=[pl.BlockSpec((1, 32, 128), lambda b, s: (b, 0, 0)),
                   rng_spec] + [rng_spec] * NDRAW,
        out_shape=[jax.ShapeDtypeStruct((B, 32, 128), jnp.float32),
                   rng_shape] + [rng_shape] * NDRAW,
        scratch_shapes=[pltpu.VMEM((32, 128), jnp.float32)],
    )(st, tt)


_ROTP = tuple([15] + list(range(15)))
_ROTM = tuple(list(range(1, 16)) + [0])


def _sc_loop_body(d_hbm, u_hbm, *rest):
    o_hbms = rest[:NDRAW]
    out_hbm, uv, ov, dv, sem_ud, sem_off = rest[NDRAW:]
    wid = lax.axis_index("s") * 2 + lax.axis_index("c")
    b = wid // (NW // 2)
    h0 = (wid - b * (NW // 2)) * RPW

    copies = []
    for dr in range(NDRAW):
        for comp in range(2):
            src = o_hbms[dr].at[pl.ds((b * 2 + comp) * HW + h0 * W, RPW * W)]
            dst = ov.at[pl.ds((dr * 2 + comp) * RPW * W, RPW * W)]
            copies.append(pltpu.async_copy(src, dst, sem_off))
    ud = []
    for comp in range(2):
        src = u_hbm.at[pl.ds((b * 2 + comp) * HW + h0 * W, RPW * W)]
        ud.append(pltpu.async_copy(src, uv.at[pl.ds(comp * RPW * W, RPW * W)], sem_ud))
    ud.append(pltpu.async_copy(d_hbm.at[pl.ds(b * HW, HW)], dv, sem_ud))
    for cp in ud:
        cp.wait()

    iota = lax.iota(jnp.int32, LANES)
    nchunk = W // LANES
    rotp_idx = (iota + (LANES - 1)) & (LANES - 1)
    rotm_idx = (iota + 1) & (LANES - 1)
    is0 = iota == 0
    is15 = iota == LANES - 1

    def lg(idx):
        return plsc.load_gather(dv, [idx])

    def rnd_int(v):
        return ((v + MAGIC) - MAGIC).astype(jnp.int32)

    def rot(v, idx):
        return v.at[idx].get(mode="promise_in_bounds")

    waited_offs = False
    for r in range(RPW):
        ys, xs, ds = [], [], []
        for i in range(nchunk):
            y = uv[pl.ds(r * W + i * LANES, LANES)] * float(H - 1)
            x = uv[pl.ds(RPW * W + r * W + i * LANES, LANES)] * float(W - 1)
            ys.append(y)
            xs.append(x)
            ds.append(lg(rnd_int(y) * W + rnd_int(x)))

        for t in range(ITERATIONS):
            for dirn in (1, -1):
                if dirn == 1:
                    ry = [rot(v, rotp_idx) for v in ys]
                    rx = [rot(v, rotp_idx) for v in xs]
                    rd = [rot(v, rotp_idx) for v in ds]
                    sy = [jnp.where(is0, ry[i - 1], ry[i]) for i in range(nchunk)]
                    sx = [jnp.where(is0, rx[i - 1], rx[i]) for i in range(nchunk)]
                    sd = [jnp.where(is0, rd[i - 1], rd[i]) for i in range(nchunk)]
                else:
                    ry = [rot(v, rotm_idx) for v in ys]
                    rx = [rot(v, rotm_idx) for v in xs]
                    rd = [rot(v, rotm_idx) for v in ds]
                    sy = [jnp.where(is15, ry[(i + 1) % nchunk], ry[i])
                          for i in range(nchunk)]
                    sx = [jnp.where(is15, rx[(i + 1) % nchunk], rx[i])
                          for i in range(nchunk)]
                    sd = [jnp.where(is15, rd[(i + 1) % nchunk], rd[i])
                          for i in range(nchunk)]
                for i in range(nchunk):
                    m = sd[i] < ds[i]
                    ys[i] = jnp.where(m, sy[i], ys[i])
                    xs[i] = jnp.where(m, sx[i], xs[i])
                    ds[i] = jnp.where(m, sd[i], ds[i])

            if not waited_offs:
                for cp in copies:
                    cp.wait()
                waited_offs = True
            for s in range(RADIUS):
                dr = t * RADIUS + s
                sc = 2.0 ** (-s)
                for i in range(nchunk):
                    ob = dr * 2 * RPW * W + r * W + i * LANES
                    offy = ov[pl.ds(ob, LANES)] * sc
                    offx = ov[pl.ds(ob + RPW * W, LANES)] * sc
                    cy = jnp.minimum(jnp.maximum(ys[i] + offy, 0.0), float(H - 1))
                    cx = jnp.minimum(jnp.maximum(xs[i] + offx, 0.0), float(W - 1))
                    cd = lg(rnd_int(cy) * W + rnd_int(cx))
                    m = cd < ds[i]
                    ys[i] = jnp.where(m, cy, ys[i])
                    xs[i] = jnp.where(m, cx, xs[i])
                    ds[i] = jnp.where(m, cd, ds[i])

        for i in range(nchunk):
            uv[pl.ds(r * W + i * LANES, LANES)] = ys[i]
            uv[pl.ds(RPW * W + r * W + i * LANES, LANES)] = xs[i]

    for comp in range(2):
        pltpu.sync_copy(uv.at[pl.ds(comp * RPW * W, RPW * W)],
                        out_hbm.at[pl.ds((b * 2 + comp) * HW + h0 * W, RPW * W)])


def _sc_loop(d_flat, u_flat, offs_list):
    B = 2
    mesh = plsc.VectorSubcoreMesh(core_axis_name="c", subcore_axis_name="s")
    fn = functools.partial(
        pl.kernel,
        mesh=mesh,
        out_type=jax.ShapeDtypeStruct((B * 2 * HW,), jnp.float32),
        scratch_types=[
            pltpu.VMEM((2 * RPW * W,), jnp.float32),
            pltpu.VMEM((NDRAW * 2 * RPW * W,), jnp.float32),
            pltpu.VMEM((HW,), jnp.float32),
            pltpu.SemaphoreType.DMA,
            pltpu.SemaphoreType.DMA,
        ],
        compiler_params=pltpu.CompilerParams(needs_layout_passes=False),
    )(_sc_loop_body)
    return fn(d_flat, u_flat, *offs_list)


_K0 = (64467757, 2916123636)
_KS = [[2451885785, 2215112154], [2477523575, 3040475525],
       [3288317168, 3869482587], [3554626980, 3142212981],
       [1115580475, 397968394], [3965541470, 1466314410],
       [1329917820, 631477198], [3389937870, 4222981018],
       [845657194, 2085162261], [2019228077, 1846897043],
       [1878397639, 3912187480], [3118403341, 2122305751]]


def kernel(source, target):
    B, C, _, _ = source.shape
    dfield, u, *offs = _dfield(source, target)
    out_flat = _sc_loop(dfield.reshape(-1), u.reshape(-1),
                        [o.reshape(-1) for o in offs])
    return out_flat.reshape(B, 2, H, W)


# speedup vs baseline: 2.5061x; 1.0608x over previous
import functools

import jax
import jax.numpy as jnp
import numpy as np
from jax import lax
from jax.experimental import pallas as pl
from jax.experimental.pallas import tpu as pltpu
from jax.experimental.pallas import tpu_sc as plsc

H = 64
W = 64
HW = H * W
ITERATIONS = 3
RADIUS = 4
NDRAW = ITERATIONS * RADIUS
NW = 32
RPW = (2 * H) // NW
LANES = 16
MAGIC = 8388608.0


_HB = 16

_ROT_A = (13, 15, 26, 6)
_ROT_B = (17, 29, 16, 24)
_LO = np.float32(np.nextafter(np.float32(-1.0), np.float32(0.0)))
_SPAN = np.float32(np.float32(1.0) - _LO)
_SQRT2 = np.float32(np.sqrt(2.0))


def _rotl(x, r):
    return (x << np.uint32(r)) | (x >> np.uint32(32 - r))


def _threefry_bits(k0, k1, x1):
    ks0 = jnp.asarray(k0, dtype=jnp.uint32)
    ks1 = jnp.asarray(k1, dtype=jnp.uint32)
    ks2 = ks0 ^ ks1 ^ np.uint32(0x1BD11BDA)
    ks = (ks0, ks1, ks2)
    x0 = jnp.broadcast_to(ks0, x1.shape)
    x1 = x1 + ks1
    for i in range(5):
        for r in (_ROT_A if i % 2 == 0 else _ROT_B):
            x0 = x0 + x1
            x1 = _rotl(x1, r)
            x1 = x1 ^ x0
        x0 = x0 + ks[(i + 1) % 3]
        x1 = x1 + ks[(i + 2) % 3] + np.uint32(i + 1)
    return x0 ^ x1


def _bits_to_unit(bits):
    f = lax.bitcast_convert_type((bits >> np.uint32(9)) | np.uint32(0x3F800000),
                                 jnp.float32)
    return f - 1.0


def _erfinv(x):
    w = -jnp.log((1.0 - x) * (1.0 + x))
    w1 = w - 2.5
    p1 = jnp.full_like(x, np.float32(2.81022636e-08))
    for c in (3.43273939e-07, -3.5233877e-06, -4.39150654e-06, 0.00021858087,
              -0.00125372503, -0.00417768164, 0.246640727, 1.50140941):
        p1 = np.float32(c) + p1 * w1
    w2 = jnp.sqrt(w) - 3.0
    p2 = jnp.full_like(x, np.float32(-0.000200214257))
    for c in (0.000100950558, 0.00134934322, -0.00367342844, 0.00573950773,
              -0.0076224613, 0.00943887047, 1.00167406, 2.83297682):
        p2 = np.float32(c) + p2 * w2
    return jnp.where(w < 5.0, p1, p2) * x


def _dfield_body(s_ref, t_ref, o_ref, u_ref, off_ref, e2_ref):
    b = pl.program_id(0)
    s = pl.program_id(1)
    ns = pl.num_programs(1)

    k = b * ns + s
    rowi = lax.broadcasted_iota(jnp.int32, (16, 128), 0)
    coli = lax.broadcasted_iota(jnp.int32, (16, 128), 1)
    iv = (rowi * 128 + coli + k * 2048).astype(jnp.uint32)
    u_ref[...] = jnp.maximum(0.0, _bits_to_unit(_threefry_bits(_K0[0], _K0[1], iv)))
    comp_pat = ((rowi % 4) >= 2).astype(jnp.int32)
    r_pat = ((rowi % 4) % 2) * 2 + coli // 64
    x_pat = coli & 63
    band = rowi // 4

    def _bandsel(vals):
        out = jnp.full((16, 128), np.uint32(vals[3]), dtype=jnp.uint32)
        for bb in (2, 1, 0):
            out = jnp.where(band == bb, np.uint32(vals[bb]), out)
        return out

    for j in range(NDRAW):
        dr0 = (j % 3) * 4
        w_id = k * 4 + (j // 3)
        b_w = w_id // 16
        h0_w = (w_id % 16) * 4
        k0c = _bandsel([_KS[dr0 + bb][0] for bb in range(4)])
        k1c = _bandsel([_KS[dr0 + bb][1] for bb in range(4)])
        f = ((b_w * 2 + comp_pat) * 4096
             + (h0_w + r_pat) * 64 + x_pat).astype(jnp.uint32)
        un = _bits_to_unit(_threefry_bits(k0c, k1c, f))
        v = jnp.maximum(_LO, un * _SPAN + _LO)
        off_ref[pl.ds(j * 16, 16), :] = _SQRT2 * _erfinv(v)

    hb = 32 // ns
    diff = s_ref[0] - t_ref[0]
    e2_ref[pl.ds(s * hb, hb), :] = jnp.sum(diff * diff, axis=-1)

    @pl.when(s == ns - 1)
    def _():
        e2 = e2_ref[...]
        rowi = lax.broadcasted_iota(jnp.int32, (32, 128), 0)
        coli = lax.broadcasted_iota(jnp.int32, (32, 128), 1)
        xi = coli & (W - 1)
        hi = coli >= W
        left = jnp.where(xi >= 1, jnp.roll(e2, 1, axis=1), 0.0)
        right = jnp.where(xi <= W - 2, jnp.roll(e2, -1, axis=1), 0.0)
        sx = e2 + left + right
        upA = jnp.roll(sx, W, axis=1)
        upC = jnp.roll(jnp.roll(sx, 1, axis=0), -W, axis=1)
        up = jnp.where(hi, upA, jnp.where(rowi >= 1, upC, 0.0))
        dnA = jnp.roll(sx, -W, axis=1)
        dnC = jnp.roll(jnp.roll(sx, -1, axis=0), W, axis=1)
        dn = jnp.where(hi, jnp.where(rowi <= 30, dnC, 0.0), dnA)
        o_ref[0] = sx + up + dn


def _dfield(source, target):
    B, C, _, _ = source.shape
    ns = 4
    st = jnp.transpose(source, (0, 2, 3, 1)).reshape(B, 32, 2 * W, C)
    tt = jnp.transpose(target, (0, 2, 3, 1)).reshape(B, 32, 2 * W, C)
    return pl.pallas_call(
        _dfield_body,
        grid=(B, ns),
        in_specs=[
            pl.BlockSpec((1, 32 // ns, 2 * W, C), lambda b, s: (b, s, 0, 0)),
            pl.BlockSpec((1, 32 // ns, 2 * W, C), lambda b, s: (b, s, 0, 0)),
        ],
        out_specs=[pl.BlockSpec((1, 32, 128), lambda b, s: (b, 0, 0)),
                   pl.BlockSpec((16, 128), lambda b, s: (b * 4 + s, 0)),
                   pl.BlockSpec((16 * NDRAW, 128), lambda b, s: (b * 4 + s, 0))],
        out_shape=[jax.ShapeDtypeStruct((B, 32, 128), jnp.float32),
                   jax.ShapeDtypeStruct((128, 128), jnp.float32),
                   jax.ShapeDtypeStruct((128 * NDRAW, 128), jnp.float32)],
        scratch_shapes=[pltpu.VMEM((32, 128), jnp.float32)],
    )(st, tt)


_ROTP = tuple([15] + list(range(15)))
_ROTM = tuple(list(range(1, 16)) + [0])


def _sc_loop_body(d_hbm, u_hbm, off_hbm, out_hbm, uv, ov, dv, sem_ud, sem_off):
    wid = lax.axis_index("s") * 2 + lax.axis_index("c")
    b = wid // (NW // 2)
    h0 = (wid - b * (NW // 2)) * RPW

    copies = [pltpu.async_copy(
        off_hbm.at[pl.ds(wid * (NDRAW * 2 * RPW * W), NDRAW * 2 * RPW * W)],
        ov, sem_off)]
    ud = []
    for comp in range(2):
        src = u_hbm.at[pl.ds((b * 2 + comp) * HW + h0 * W, RPW * W)]
        ud.append(pltpu.async_copy(src, uv.at[pl.ds(comp * RPW * W, RPW * W)], sem_ud))
    ud.append(pltpu.async_copy(d_hbm.at[pl.ds(b * HW, HW)], dv, sem_ud))
    for cp in ud:
        cp.wait()

    iota = lax.iota(jnp.int32, LANES)
    nchunk = W // LANES
    rotp_idx = (iota + (LANES - 1)) & (LANES - 1)
    rotm_idx = (iota + 1) & (LANES - 1)
    is0 = iota == 0
    is15 = iota == LANES - 1

    def lg(idx):
        return plsc.load_gather(dv, [idx])

    def rnd_int(v):
        return ((v + MAGIC) - MAGIC).astype(jnp.int32)

    def rot(v, idx):
        return v.at[idx].get(mode="promise_in_bounds")

    for cp in copies:
        cp.wait()

    def row_body(r, carry):
        ys, xs, ds = [], [], []
        for i in range(nchunk):
            y = plsc.load_gather(uv, [iota + (r * W + i * LANES)]) * float(H - 1)
            x = plsc.load_gather(
                uv, [iota + (RPW * W + r * W + i * LANES)]) * float(W - 1)
            ys.append(y)
            xs.append(x)
            ds.append(lg(rnd_int(y) * W + rnd_int(x)))

        for t in range(ITERATIONS):
            for dirn in (1, -1):
                if dirn == 1:
                    ry = [rot(v, rotp_idx) for v in ys]
                    rx = [rot(v, rotp_idx) for v in xs]
                    rd = [rot(v, rotp_idx) for v in ds]
                    sy = [jnp.where(is0, ry[i - 1], ry[i]) for i in range(nchunk)]
                    sx = [jnp.where(is0, rx[i - 1], rx[i]) for i in range(nchunk)]
                    sd = [jnp.where(is0, rd[i - 1], rd[i]) for i in range(nchunk)]
                else:
                    ry = [rot(v, rotm_idx) for v in ys]
                    rx = [rot(v, rotm_idx) for v in xs]
                    rd = [rot(v, rotm_idx) for v in ds]
                    sy = [jnp.where(is15, ry[(i + 1) % nchunk], ry[i])
                          for i in range(nchunk)]
                    sx = [jnp.where(is15, rx[(i + 1) % nchunk], rx[i])
                          for i in range(nchunk)]
                    sd = [jnp.where(is15, rd[(i + 1) % nchunk], rd[i])
                          for i in range(nchunk)]
                for i in range(nchunk):
                    m = sd[i] < ds[i]
                    ys[i] = jnp.where(m, sy[i], ys[i])
                    xs[i] = jnp.where(m, sx[i], xs[i])
                    ds[i] = jnp.where(m, sd[i], ds[i])

            for s in range(RADIUS):
                dr = t * RADIUS + s
                sc = 2.0 ** (-s)
                for i in range(nchunk):
                    ob = dr * 2 * RPW * W + r * W + i * LANES
                    offy = plsc.load_gather(ov, [iota + ob]) * sc
                    offx = plsc.load_gather(ov, [iota + (ob + RPW * W)]) * sc
                    cy = jnp.minimum(jnp.maximum(ys[i] + offy, 0.0), float(H - 1))
                    cx = jnp.minimum(jnp.maximum(xs[i] + offx, 0.0), float(W - 1))
                    cd = lg(rnd_int(cy) * W + rnd_int(cx))
                    m = cd < ds[i]
                    ys[i] = jnp.where(m, cy, ys[i])
                    xs[i] = jnp.where(m, cx, xs[i])
                    ds[i] = jnp.where(m, cd, ds[i])

        for i in range(nchunk):
            plsc.store_scatter(uv, [iota + (r * W + i * LANES)], ys[i])
            plsc.store_scatter(uv, [iota + (RPW * W + r * W + i * LANES)], xs[i])
        return carry

    lax.fori_loop(0, RPW, row_body, 0)

    for comp in range(2):
        pltpu.sync_copy(uv.at[pl.ds(comp * RPW * W, RPW * W)],
                        out_hbm.at[pl.ds((b * 2 + comp) * HW + h0 * W, RPW * W)])


def _sc_loop(d_flat, u_flat, off_flat):
    B = 2
    mesh = plsc.VectorSubcoreMesh(core_axis_name="c", subcore_axis_name="s")
    fn = functools.partial(
        pl.kernel,
        mesh=mesh,
        out_type=jax.ShapeDtypeStruct((B * 2 * HW,), jnp.float32),
        scratch_types=[
            pltpu.VMEM((2 * RPW * W,), jnp.float32),
            pltpu.VMEM((NDRAW * 2 * RPW * W,), jnp.float32),
            pltpu.VMEM((HW,), jnp.float32),
            pltpu.SemaphoreType.DMA,
            pltpu.SemaphoreType.DMA,
        ],
        compiler_params=pltpu.CompilerParams(needs_layout_passes=False),
    )(_sc_loop_body)
    return fn(d_flat, u_flat, off_flat)


_K0 = (64467757, 2916123636)
_KS = [[2451885785, 2215112154], [2477523575, 3040475525],
       [3288317168, 3869482587], [3554626980, 3142212981],
       [1115580475, 397968394], [3965541470, 1466314410],
       [1329917820, 631477198], [3389937870, 4222981018],
       [845657194, 2085162261], [2019228077, 1846897043],
       [1878397639, 3912187480], [3118403341, 2122305751]]


def kernel(source, target):
    B, C, _, _ = source.shape
    dfield, u, offs = _dfield(source, target)
    out_flat = _sc_loop(dfield.reshape(-1), u.reshape(-1), offs.reshape(-1))
    return out_flat.reshape(B, 2, H, W)
